# baseline (speedup 1.0000x reference)
import jax
import jax.numpy as jnp
from jax import lax
from jax.experimental import pallas as pl
from jax.experimental.pallas import tpu as pltpu
from jax.experimental.pallas import tpu_sc as plsc

N = 50000
E = 800000
DIM = 64
HALF = 32

SB = 128
E_PAD = 819200
SUBB = E_PAD // SB
HQ = E_PAD // 4
NSC = 2
NSUB = 16
R = SUBB // NSUB
CH = 16
N_PAD = 50048
NODES_PER_SUB = N_PAD // NSUB

BK = 6400
BQ = BK // 4


def _mlp_body(rbft_ref, w1_ref, b1_ref, w2_ref, b2_ref, lo_ref, hi_ref):
    h = lax.dot_general(rbft_ref[...], w1_ref[...], (((0,), (0,)), ((), ())),
                        preferred_element_type=jnp.float32) + b1_ref[...]
    bx = 0.5 * h
    act = jnp.where(bx > 14.0, h,
                    2.0 * jnp.log1p(jnp.exp(jnp.minimum(bx, 14.0))))
    h2 = lax.dot_general(act, w2_ref[...], (((1,), (0,)), ((), ())),
                         preferred_element_type=jnp.float32) + b2_ref[...]
    lo_ref[...] = jnp.concatenate(
        [h2[t * BQ:(t + 1) * BQ, :HALF] for t in range(4)], axis=1)
    hi_ref[...] = jnp.concatenate(
        [h2[t * BQ:(t + 1) * BQ, HALF:] for t in range(4)], axis=1)


def _edge_mlp(rbf, W1, b1, W2, b2):
    grid = (E // BK,)
    return pl.pallas_call(
        _mlp_body,
        grid=grid,
        in_specs=[
            pl.BlockSpec((DIM, BK), lambda i: (0, i)),
            pl.BlockSpec((DIM, DIM), lambda i: (0, 0)),
            pl.BlockSpec((1, DIM), lambda i: (0, 0)),
            pl.BlockSpec((DIM, DIM), lambda i: (0, 0)),
            pl.BlockSpec((1, DIM), lambda i: (0, 0)),
        ],
        out_specs=[
            pl.BlockSpec((BQ, 4 * HALF), lambda i: (i, 0)),
            pl.BlockSpec((BQ, 4 * HALF), lambda i: (i, 0)),
        ],
        out_shape=[
            jax.ShapeDtypeStruct((HQ, 4 * HALF), jnp.float32),
            jax.ShapeDtypeStruct((HQ, 4 * HALF), jnp.float32),
        ],
    )(rbf.T, W1, b1.reshape(1, DIM), W2, b2.reshape(1, DIM))


def _mult_store(msg, xr, hv, coff):
    @pl.loop(0, SB // 4, unroll=4)
    def _(q):
        for t in range(4):
            k = q * 4 + t
            msg[k, pl.ds(0, 16)] = xr[k, pl.ds(coff, 16)] * hv[q, pl.ds(t * 32, 16)]
            msg[k, pl.ds(16, 16)] = xr[k, pl.ds(coff + 16, 16)] * hv[q, pl.ds(t * 32 + 16, 16)]


def _sc_body(xlo_hbm, xhi_hbm, src_hbm, dst_hbm, hlo_hbm, hhi_hbm, out_hbm,
             acc, idx_s, idx_d, xr0, xr1, hv0, hv1, msg,
             sx0, sx1, sh0, sh1):
    cid = lax.axis_index("c")
    sid = lax.axis_index("s")

    zeros16 = jnp.zeros((16,), jnp.float32)

    @pl.loop(0, SB)
    def _(k):
        msg[k, pl.ds(0, 16)] = zeros16
        msg[k, pl.ds(16, 16)] = zeros16

    acc_base = sid * NODES_PER_SUB

    @pl.loop(0, 24)
    def _(i):
        pltpu.sync_copy(msg, acc.at[pl.ds(acc_base + i * SB, SB)])

    pltpu.sync_copy(msg.at[pl.ds(0, 56)],
                    acc.at[pl.ds(acc_base + 24 * SB, 56)])

    plsc.subcore_barrier()

    @pl.loop(0, R // CH)
    def _(ci):
        row0 = sid * R + ci * CH
        pltpu.sync_copy(src_hbm.at[pl.ds(row0, CH)], idx_s)
        pltpu.sync_copy(dst_hbm.at[pl.ds(row0, CH)], idx_d)

        def issue(j, xr_b, sx_b, hv_b, sh_b):
            @pl.when(cid == 0)
            def _():
                pltpu.async_copy(xlo_hbm.at[idx_s.at[j]], xr_b, sx_b)
                pltpu.async_copy(
                    hlo_hbm.at[pl.ds((row0 + j) * (SB // 4), SB // 4)],
                    hv_b, sh_b)

            @pl.when(cid == 1)
            def _():
                pltpu.async_copy(xhi_hbm.at[idx_s.at[j]], xr_b, sx_b)
                pltpu.async_copy(
                    hhi_hbm.at[pl.ds((row0 + j) * (SB // 4), SB // 4)],
                    hv_b, sh_b)

        issue(0, xr0, sx0, hv0, sh0)

        @pl.loop(0, CH // 2)
        def _(jj):
            for b in range(2):
                j = jj * 2 + b
                xr_cur, sx_cur, hv_cur, sh_cur = (
                    (xr0, sx0, hv0, sh0) if b == 0 else (xr1, sx1, hv1, sh1))
                xr_nxt, sx_nxt, hv_nxt, sh_nxt = (
                    (xr1, sx1, hv1, sh1) if b == 0 else (xr0, sx0, hv0, sh0))

                @pl.when(j + 1 < CH)
                def _():
                    issue(j + 1, xr_nxt, sx_nxt, hv_nxt, sh_nxt)

                pltpu.make_async_copy(
                    xlo_hbm.at[idx_s.at[j]], xr_cur, sx_cur).wait()
                pltpu.make_async_copy(
                    hlo_hbm.at[pl.ds(0, SB // 4)], hv_cur, sh_cur).wait()

                _mult_store(msg, xr_cur, hv_cur, 0)

                pltpu.sync_copy(msg, acc.at[idx_d.at[j]], add=True)

    plsc.subcore_barrier()

    pltpu.sync_copy(acc.at[pl.ds(acc_base, NODES_PER_SUB)],
                    out_hbm.at[cid, pl.ds(acc_base, NODES_PER_SUB)])


def _sc_aggregate(xlo, xhi, src2, dst2, hlo2, hhi2):
    mesh = plsc.VectorSubcoreMesh(core_axis_name="c", subcore_axis_name="s")
    f = pl.kernel(
        _sc_body,
        out_type=jax.ShapeDtypeStruct((NSC, N_PAD, HALF), jnp.float32),
        mesh=mesh,
        compiler_params=pltpu.CompilerParams(use_tc_tiling_on_sc=False),
        scratch_types=[
            pltpu.VMEM_SHARED((N_PAD, HALF), jnp.float32),
            pltpu.VMEM((CH, SB), jnp.int32),
            pltpu.VMEM((CH, SB), jnp.int32),
            pltpu.VMEM((SB, HALF), jnp.float32),
            pltpu.VMEM((SB, HALF), jnp.float32),
            pltpu.VMEM((SB // 4, 4 * HALF), jnp.float32),
            pltpu.VMEM((SB // 4, 4 * HALF), jnp.float32),
            pltpu.VMEM((SB, HALF), jnp.float32),
            pltpu.SemaphoreType.DMA,
            pltpu.SemaphoreType.DMA,
            pltpu.SemaphoreType.DMA,
            pltpu.SemaphoreType.DMA,
        ],
    )
    return f(xlo, xhi, src2, dst2, hlo2, hhi2)


def _pack_order(v, fill):
    vp3 = v.reshape(E // BK, 4, BQ).transpose(0, 2, 1)
    pad3 = jnp.full(((E_PAD - E) // BK, BQ, 4), fill, jnp.int32)
    return jnp.concatenate([vp3, pad3]).reshape(SUBB, SB)


def kernel(x, edge_index, rbf, W1, b1, W2, b2):
    hlo2, hhi2 = _edge_mlp(rbf, W1, b1, W2, b2)
    src2 = _pack_order(edge_index[0], 0)
    dst2 = _pack_order(edge_index[1], N)
    out = _sc_aggregate(x[:, :HALF], x[:, HALF:], src2, dst2, hlo2, hhi2)
    return jnp.concatenate([out[0, :N], out[1, :N]], axis=1)


# speedup vs baseline: 1.0380x; 1.0380x over previous
import jax
import jax.numpy as jnp
from jax import lax
from jax.experimental import pallas as pl
from jax.experimental.pallas import tpu as pltpu
from jax.experimental.pallas import tpu_sc as plsc

N = 50000
E = 800000
DIM = 64
HALF = 32

SB = 128
E_PAD = 819200
SUBB = E_PAD // SB
HQ = E_PAD // 4
NSC = 2
NSUB = 16
R = SUBB // NSUB
CH = 8
N_PAD = 50048
NODES_PER_SUB = N_PAD // NSUB

BK = 6400
BQ = BK // 4


def _mlp_body(rbft_ref, w1_ref, b1_ref, w2_ref, b2_ref, lo_ref, hi_ref):
    h = lax.dot_general(rbft_ref[...], w1_ref[...], (((0,), (0,)), ((), ())),
                        preferred_element_type=jnp.float32) + b1_ref[...]
    bx = 0.5 * h
    act = jnp.where(bx > 14.0, h,
                    2.0 * jnp.log1p(jnp.exp(jnp.minimum(bx, 14.0))))
    h2 = lax.dot_general(act, w2_ref[...], (((1,), (0,)), ((), ())),
                         preferred_element_type=jnp.float32) + b2_ref[...]
    lo_ref[...] = jnp.concatenate(
        [h2[t * BQ:(t + 1) * BQ, :HALF] for t in range(4)], axis=1)
    hi_ref[...] = jnp.concatenate(
        [h2[t * BQ:(t + 1) * BQ, HALF:] for t in range(4)], axis=1)


def _edge_mlp(rbf, W1, b1, W2, b2):
    grid = (E // BK,)
    return pl.pallas_call(
        _mlp_body,
        grid=grid,
        in_specs=[
            pl.BlockSpec((DIM, BK), lambda i: (0, i)),
            pl.BlockSpec((DIM, DIM), lambda i: (0, 0)),
            pl.BlockSpec((1, DIM), lambda i: (0, 0)),
            pl.BlockSpec((DIM, DIM), lambda i: (0, 0)),
            pl.BlockSpec((1, DIM), lambda i: (0, 0)),
        ],
        out_specs=[
            pl.BlockSpec((BQ, 4 * HALF), lambda i: (i, 0)),
            pl.BlockSpec((BQ, 4 * HALF), lambda i: (i, 0)),
        ],
        out_shape=[
            jax.ShapeDtypeStruct((HQ, 4 * HALF), jnp.float32),
            jax.ShapeDtypeStruct((HQ, 4 * HALF), jnp.float32),
        ],
    )(rbf.T, W1, b1.reshape(1, DIM), W2, b2.reshape(1, DIM))


def _mult_store(xr, hv):
    @pl.loop(0, SB // 4, unroll=4)
    def _(q):
        for t in range(4):
            k = q * 4 + t
            xr[k, pl.ds(0, 16)] = xr[k, pl.ds(0, 16)] * hv[q, pl.ds(t * 32, 16)]
            xr[k, pl.ds(16, 16)] = xr[k, pl.ds(16, 16)] * hv[q, pl.ds(t * 32 + 16, 16)]


def _sc_body(xlo_hbm, xhi_hbm, src_hbm, dst_hbm, hlo_hbm, hhi_hbm, out_hbm,
             acc, idx_s, idx_d, xr0, xr1, xr2, xr3, hv0, hv1,
             sx0, sx1, sx2, sx3, sh0, sh1):
    cid = lax.axis_index("c")
    sid = lax.axis_index("s")

    zeros16 = jnp.zeros((16,), jnp.float32)

    @pl.loop(0, SB)
    def _(k):
        xr0[k, pl.ds(0, 16)] = zeros16
        xr0[k, pl.ds(16, 16)] = zeros16

    acc_base = sid * NODES_PER_SUB

    @pl.loop(0, 24)
    def _(i):
        pltpu.sync_copy(xr0, acc.at[pl.ds(acc_base + i * SB, SB)])

    pltpu.sync_copy(xr0.at[pl.ds(0, 56)],
                    acc.at[pl.ds(acc_base + 24 * SB, 56)])

    plsc.subcore_barrier()

    xrs = (xr0, xr1, xr2, xr3)
    sxs = (sx0, sx1, sx2, sx3)
    hvs = (hv0, hv1)
    shs = (sh0, sh1)

    @pl.loop(0, R // CH)
    def _(ci):
        row0 = sid * R + ci * CH
        pltpu.sync_copy(src_hbm.at[pl.ds(row0, CH)], idx_s)
        pltpu.sync_copy(dst_hbm.at[pl.ds(row0, CH)], idx_d)

        def issue_x(j, slot):
            @pl.when(cid == 0)
            def _():
                pltpu.async_copy(xlo_hbm.at[idx_s.at[j]], xrs[slot], sxs[slot])

            @pl.when(cid == 1)
            def _():
                pltpu.async_copy(xhi_hbm.at[idx_s.at[j]], xrs[slot], sxs[slot])

        def issue_h(j, slot):
            @pl.when(cid == 0)
            def _():
                pltpu.async_copy(
                    hlo_hbm.at[pl.ds((row0 + j) * (SB // 4), SB // 4)],
                    hvs[slot], shs[slot])

            @pl.when(cid == 1)
            def _():
                pltpu.async_copy(
                    hhi_hbm.at[pl.ds((row0 + j) * (SB // 4), SB // 4)],
                    hvs[slot], shs[slot])

        issue_x(0, 0)
        issue_x(1, 1)
        issue_x(2, 2)
        issue_h(0, 0)

        @pl.loop(0, CH // 4)
        def _(jj):
            for b in range(4):
                j = jj * 4 + b
                xr_cur, sx_cur = xrs[b], sxs[b]
                hv_cur, sh_cur = hvs[b % 2], shs[b % 2]

                @pl.when(j + 3 < CH)
                def _():
                    issue_x(j + 3, (b + 3) % 4)

                @pl.when(j + 1 < CH)
                def _():
                    issue_h(j + 1, (b + 1) % 2)

                pltpu.make_async_copy(
                    xlo_hbm.at[idx_s.at[0]], xr_cur, sx_cur).wait()
                pltpu.make_async_copy(
                    hlo_hbm.at[pl.ds(0, SB // 4)], hv_cur, sh_cur).wait()

                _mult_store(xr_cur, hv_cur)

                pltpu.sync_copy(xr_cur, acc.at[idx_d.at[j]], add=True)

    plsc.subcore_barrier()

    pltpu.sync_copy(acc.at[pl.ds(acc_base, NODES_PER_SUB)],
                    out_hbm.at[cid, pl.ds(acc_base, NODES_PER_SUB)])


def _sc_aggregate(xlo, xhi, src2, dst2, hlo2, hhi2):
    mesh = plsc.VectorSubcoreMesh(core_axis_name="c", subcore_axis_name="s")
    f = pl.kernel(
        _sc_body,
        out_type=jax.ShapeDtypeStruct((NSC, N_PAD, HALF), jnp.float32),
        mesh=mesh,
        compiler_params=pltpu.CompilerParams(use_tc_tiling_on_sc=False),
        scratch_types=[
            pltpu.VMEM_SHARED((N_PAD, HALF), jnp.float32),
            pltpu.VMEM((CH, SB), jnp.int32),
            pltpu.VMEM((CH, SB), jnp.int32),
            pltpu.VMEM((SB, HALF), jnp.float32),
            pltpu.VMEM((SB, HALF), jnp.float32),
            pltpu.VMEM((SB, HALF), jnp.float32),
            pltpu.VMEM((SB, HALF), jnp.float32),
            pltpu.VMEM((SB // 4, 4 * HALF), jnp.float32),
            pltpu.VMEM((SB // 4, 4 * HALF), jnp.float32),
            pltpu.SemaphoreType.DMA,
            pltpu.SemaphoreType.DMA,
            pltpu.SemaphoreType.DMA,
            pltpu.SemaphoreType.DMA,
            pltpu.SemaphoreType.DMA,
            pltpu.SemaphoreType.DMA,
        ],
    )
    return f(xlo, xhi, src2, dst2, hlo2, hhi2)


def _pack_order(v, fill):
    vp = v.reshape(E // BK, 4, BQ).transpose(0, 2, 1).reshape(E)
    return jnp.concatenate(
        [vp, jnp.full((E_PAD - E,), fill, jnp.int32)]).reshape(SUBB, SB)


def kernel(x, edge_index, rbf, W1, b1, W2, b2):
    hlo2, hhi2 = _edge_mlp(rbf, W1, b1, W2, b2)
    src2 = _pack_order(edge_index[0], 0)
    dst2 = _pack_order(edge_index[1], N)
    out = _sc_aggregate(x[:, :HALF], x[:, HALF:], src2, dst2, hlo2, hhi2)
    return jnp.concatenate([out[0, :N], out[1, :N]], axis=1)


# speedup vs baseline: 1.0771x; 1.0377x over previous
import jax
import jax.numpy as jnp
from jax import lax
from jax.experimental import pallas as pl
from jax.experimental.pallas import tpu as pltpu
from jax.experimental.pallas import tpu_sc as plsc

N = 50000
E = 800000
DIM = 64
HALF = 32

SB = 128
E_PAD = 819200
SUBB = E_PAD // SB
HQ = E_PAD // 4
NSC = 2
NSUB = 16
R = SUBB // NSUB
CH = 16
N_PAD = 50048
NODES_PER_SUB = N_PAD // NSUB

BK = 6400
BQ = BK // 4


def _mlp_body(rbft_ref, w1_ref, b1_ref, w2_ref, b2_ref, lo_ref, hi_ref):
    h = lax.dot_general(rbft_ref[...], w1_ref[...], (((0,), (0,)), ((), ())),
                        preferred_element_type=jnp.float32) + b1_ref[...]
    bx = 0.5 * h
    act = jnp.where(bx > 14.0, h,
                    2.0 * jnp.log1p(jnp.exp(jnp.minimum(bx, 14.0))))
    h2 = lax.dot_general(act, w2_ref[...], (((1,), (0,)), ((), ())),
                         preferred_element_type=jnp.float32) + b2_ref[...]
    lo_ref[...] = jnp.concatenate(
        [h2[t * BQ:(t + 1) * BQ, :HALF] for t in range(4)], axis=1)
    hi_ref[...] = jnp.concatenate(
        [h2[t * BQ:(t + 1) * BQ, HALF:] for t in range(4)], axis=1)


def _edge_mlp(rbf, W1, b1, W2, b2):
    grid = (E // BK,)
    return pl.pallas_call(
        _mlp_body,
        grid=grid,
        in_specs=[
            pl.BlockSpec((DIM, BK), lambda i: (0, i)),
            pl.BlockSpec((DIM, DIM), lambda i: (0, 0)),
            pl.BlockSpec((1, DIM), lambda i: (0, 0)),
            pl.BlockSpec((DIM, DIM), lambda i: (0, 0)),
            pl.BlockSpec((1, DIM), lambda i: (0, 0)),
        ],
        out_specs=[
            pl.BlockSpec((BQ, 4 * HALF), lambda i: (i, 0)),
            pl.BlockSpec((BQ, 4 * HALF), lambda i: (i, 0)),
        ],
        out_shape=[
            jax.ShapeDtypeStruct((HQ, 4 * HALF), jnp.float32),
            jax.ShapeDtypeStruct((HQ, 4 * HALF), jnp.float32),
        ],
    )(rbf.T, W1, b1.reshape(1, DIM), W2, b2.reshape(1, DIM))


def _mult_store(xr, hv):
    @pl.loop(0, SB // 4, unroll=4)
    def _(q):
        for t in range(4):
            k = q * 4 + t
            xr[k, pl.ds(0, 16)] = xr[k, pl.ds(0, 16)] * hv[q, pl.ds(t * 32, 16)]
            xr[k, pl.ds(16, 16)] = xr[k, pl.ds(16, 16)] * hv[q, pl.ds(t * 32 + 16, 16)]


def _sc_body(xlo_hbm, xhi_hbm, src_hbm, dst_hbm, hlo_hbm, hhi_hbm, out_hbm,
             acc, idx_s, idx_d, xr0, xr1, hv0, hv1,
             sx0, sx1, sh0, sh1):
    cid = lax.axis_index("c")
    sid = lax.axis_index("s")

    zeros16 = jnp.zeros((16,), jnp.float32)

    @pl.loop(0, SB)
    def _(k):
        xr0[k, pl.ds(0, 16)] = zeros16
        xr0[k, pl.ds(16, 16)] = zeros16

    acc_base = sid * NODES_PER_SUB

    @pl.loop(0, 24)
    def _(i):
        pltpu.sync_copy(xr0, acc.at[pl.ds(acc_base + i * SB, SB)])

    pltpu.sync_copy(xr0.at[pl.ds(0, 56)],
                    acc.at[pl.ds(acc_base + 24 * SB, 56)])

    plsc.subcore_barrier()

    xrs = (xr0, xr1)
    sxs = (sx0, sx1)
    hvs = (hv0, hv1)
    shs = (sh0, sh1)

    @pl.loop(0, R // CH)
    def _(ci):
        row0 = sid * R + ci * CH
        pltpu.sync_copy(src_hbm.at[pl.ds(row0, CH)], idx_s)
        pltpu.sync_copy(dst_hbm.at[pl.ds(row0, CH)], idx_d)

        def issue_x(j, slot):
            @pl.when(cid == 0)
            def _():
                pltpu.async_copy(xlo_hbm.at[idx_s.at[j]], xrs[slot], sxs[slot])

            @pl.when(cid == 1)
            def _():
                pltpu.async_copy(xhi_hbm.at[idx_s.at[j]], xrs[slot], sxs[slot])

        def issue_h(j, slot):
            @pl.when(cid == 0)
            def _():
                pltpu.async_copy(
                    hlo_hbm.at[pl.ds((row0 + j) * (SB // 4), SB // 4)],
                    hvs[slot], shs[slot])

            @pl.when(cid == 1)
            def _():
                pltpu.async_copy(
                    hhi_hbm.at[pl.ds((row0 + j) * (SB // 4), SB // 4)],
                    hvs[slot], shs[slot])

        issue_x(0, 0)
        issue_h(0, 0)

        @pl.loop(0, CH // 2)
        def _(jj):
            for b in range(2):
                j = jj * 2 + b
                xr_cur, sx_cur = xrs[b], sxs[b]
                hv_cur, sh_cur = hvs[b], shs[b]

                @pl.when(j + 1 < CH)
                def _():
                    issue_x(j + 1, 1 - b)
                    issue_h(j + 1, 1 - b)

                pltpu.make_async_copy(
                    xlo_hbm.at[idx_s.at[j]], xr_cur, sx_cur).wait()
                pltpu.make_async_copy(
                    hlo_hbm.at[pl.ds(0, SB // 4)], hv_cur, sh_cur).wait()

                _mult_store(xr_cur, hv_cur)

                pltpu.sync_copy(xr_cur, acc.at[idx_d.at[j]], add=True)

    plsc.subcore_barrier()

    pltpu.sync_copy(acc.at[pl.ds(acc_base, NODES_PER_SUB)],
                    out_hbm.at[cid, pl.ds(acc_base, NODES_PER_SUB)])


def _sc_aggregate(xlo, xhi, src2, dst2, hlo2, hhi2):
    mesh = plsc.VectorSubcoreMesh(core_axis_name="c", subcore_axis_name="s")
    f = pl.kernel(
        _sc_body,
        out_type=jax.ShapeDtypeStruct((NSC, N_PAD, HALF), jnp.float32),
        mesh=mesh,
        compiler_params=pltpu.CompilerParams(use_tc_tiling_on_sc=False),
        scratch_types=[
            pltpu.VMEM_SHARED((N_PAD, HALF), jnp.float32),
            pltpu.VMEM((CH, SB), jnp.int32),
            pltpu.VMEM((CH, SB), jnp.int32),
            pltpu.VMEM((SB, HALF), jnp.float32),
            pltpu.VMEM((SB, HALF), jnp.float32),
            pltpu.VMEM((SB // 4, 4 * HALF), jnp.float32),
            pltpu.VMEM((SB // 4, 4 * HALF), jnp.float32),
            pltpu.SemaphoreType.DMA,
            pltpu.SemaphoreType.DMA,
            pltpu.SemaphoreType.DMA,
            pltpu.SemaphoreType.DMA,
        ],
    )
    return f(xlo, xhi, src2, dst2, hlo2, hhi2)


def _pack_order(v, fill):
    vp = v.reshape(E // BK, 4, BQ).transpose(0, 2, 1).reshape(E)
    return jnp.concatenate(
        [vp, jnp.full((E_PAD - E,), fill, jnp.int32)]).reshape(SUBB, SB)


def kernel(x, edge_index, rbf, W1, b1, W2, b2):
    hlo2, hhi2 = _edge_mlp(rbf, W1, b1, W2, b2)
    src2 = _pack_order(edge_index[0], 0)
    dst2 = _pack_order(edge_index[1], N)
    out = _sc_aggregate(x[:, :HALF], x[:, HALF:], src2, dst2, hlo2, hhi2)
    return jnp.concatenate([out[0, :N], out[1, :N]], axis=1)
